# Initial kernel scaffold; baseline (speedup 1.0000x reference)
import jax
import jax.numpy as jnp
from jax.experimental import pallas as pl


def kernel(x, edge_index, W1_l, b1_l, W1_r, W2_l, b2_l, W2_r, Wl1, bl1, Wl2, bl2):
    raise NotImplementedError("write your pallas kernel here")


# speedup vs baseline: 4.0814x; 4.0814x over previous
import jax
import jax.numpy as jnp
from jax import lax
from jax.experimental import pallas as pl
from jax.experimental.pallas import tpu as pltpu
from jax.experimental.pallas import tpu_sc as plsc

N = 10000
E = 320000
D = 128

NC = 2
NS = 16
NW = NC * NS

EDGES_PER_TILE = E // NW
TILE_PAD = 10240
CHUNK = 80
CHUNKS = TILE_PAD // CHUNK
QCHUNKS = 32
PHASES = CHUNKS // QCHUNKS
N_ACC = 10240
TRASH = N
STRIPE = N_ACC // NS
ZROWS = 16
LAST_OUT = N - (NS - 1) * STRIPE

f32 = jnp.float32
i32 = jnp.int32


def _fill2d(ref, rows, cols, value):
    vals = jnp.full((16,), value, f32)

    def body(i, _):
        def body2(j, __):
            ref[i, pl.ds(j * 16, 16)] = vals
            return 0
        return lax.fori_loop(0, cols // 16, body2, 0)

    lax.fori_loop(0, rows, body, 0)


def _make_agg():
    mesh = plsc.VectorSubcoreMesh(core_axis_name="c", subcore_axis_name="s")
    out_type = [jax.ShapeDtypeStruct((NC, N, D), f32)]
    scratch = [
        pltpu.VMEM((QCHUNKS, CHUNK), i32),
        pltpu.VMEM((QCHUNKS, CHUNK), i32),
        pltpu.VMEM((CHUNK, D), f32),
        pltpu.VMEM((ZROWS, D), f32),
        pltpu.VMEM_SHARED((N_ACC, D), f32),
        pltpu.SemaphoreType.DMA,
    ]

    def body(table, src2d, dst2d, sum_out, src_v, dst_v, rows_v, zbuf,
             accum, sem):
        c = lax.axis_index("c")
        s = lax.axis_index("s")
        w = c * NS + s

        _fill2d(zbuf, ZROWS, D, 0.0)
        row0 = s * STRIPE
        for k in range(STRIPE // ZROWS):
            pltpu.sync_copy(zbuf, accum.at[pl.ds(row0 + k * ZROWS, ZROWS)])
        plsc.subcore_barrier()

        def phase(p, _):
            base = w * CHUNKS + p * QCHUNKS
            pltpu.sync_copy(src2d.at[pl.ds(base, QCHUNKS)], src_v)
            pltpu.sync_copy(dst2d.at[pl.ds(base, QCHUNKS)], dst_v)

            def chunk(j, __):
                pltpu.async_copy(table.at[src_v.at[j]], rows_v, sem).wait()
                pltpu.sync_copy(rows_v, accum.at[dst_v.at[j]], add=True)
                return 0

            return lax.fori_loop(0, QCHUNKS, chunk, 0)

        lax.fori_loop(0, PHASES, phase, 0)
        plsc.subcore_barrier()

        @pl.when(s < NS - 1)
        def _():
            pltpu.sync_copy(accum.at[pl.ds(row0, STRIPE)],
                            sum_out.at[c, pl.ds(row0, STRIPE)])

        @pl.when(s == NS - 1)
        def _():
            pltpu.sync_copy(accum.at[pl.ds(row0, LAST_OUT)],
                            sum_out.at[c, pl.ds(row0, LAST_OUT)])

    return pl.kernel(body, out_type=out_type, mesh=mesh,
                     scratch_types=scratch)


E_PAD = NW * TILE_PAD
CBS = 1024


def _cnt_body(dst_ref, cnt_ref):
    step = pl.program_id(0)

    @pl.when(step == 0)
    def _():
        cnt_ref[...] = jnp.zeros((128, 128), f32)

    d = dst_ref[...]
    lo = d & 127
    hi = d >> 7
    lane = lax.iota(i32, 128).reshape(1, 128)
    mlo = (lo == lane).astype(f32)
    mhi = (hi == lane).astype(f32)
    cnt_ref[...] += lax.dot_general(
        mhi, mlo, (((0,), (0,)), ((), ())), preferred_element_type=f32)


_count_tc = pl.pallas_call(
    _cnt_body,
    grid=(E_PAD // CBS,),
    in_specs=[pl.BlockSpec((CBS, 1), lambda i: (i, 0))],
    out_specs=pl.BlockSpec((128, 128), lambda i: (0, 0)),
    out_shape=jax.ShapeDtypeStruct((128, 128), f32),
)


_agg = _make_agg()


def _first(res):
    return res[0] if isinstance(res, (list, tuple)) else res


def _dotT(a, b):
    return lax.dot_general(a, b, (((1,), (1,)), ((), ())),
                           preferred_element_type=f32)


def _tc1(s1p_ref, rc_ref, x_ref, w1l_ref, b1l_ref, w1r_ref, w2l_ref,
         w2r_ref, y2_ref, r2_ref):
    rc = rc_ref[...]
    mean1 = (s1p_ref[0] + s1p_ref[1]) * rc
    h1 = _dotT(mean1, w1l_ref[...]) + b1l_ref[...] + _dotT(x_ref[...],
                                                           w1r_ref[...])
    h1 = jnp.maximum(h1, 0.0)
    y2_ref[...] = _dotT(h1, w2l_ref[...])
    r2_ref[...] = _dotT(h1, w2r_ref[...])


def _tc2(s2p_ref, rc_ref, r2_ref, b2l_ref, wl1_ref, bl1_ref, wl2_ref,
         bl2_ref, out_ref):
    rc = rc_ref[...]
    mean2 = (s2p_ref[0] + s2p_ref[1]) * rc
    h2 = jnp.maximum(mean2 + b2l_ref[...] + r2_ref[...], 0.0)
    h3 = _dotT(h2, wl1_ref[...]) + bl1_ref[...]
    out_ref[...] = _dotT(h3, wl2_ref[...]) + bl2_ref[...]


@jax.jit
def kernel(x, edge_index, W1_l, b1_l, W1_r, W2_l, b2_l, W2_r, Wl1, bl1,
           Wl2, bl2):
    pad = TILE_PAD - EDGES_PER_TILE
    src2d = jnp.pad(edge_index[0].reshape(NW, EDGES_PER_TILE),
                    ((0, 0), (0, pad)),
                    constant_values=0).reshape(NW * CHUNKS, CHUNK)
    dst2d = jnp.pad(edge_index[1].reshape(NW, EDGES_PER_TILE),
                    ((0, 0), (0, pad)),
                    constant_values=TRASH).reshape(NW * CHUNKS, CHUNK)

    s1p = _first(_agg(x, src2d, dst2d))
    cnt = _count_tc(dst2d.reshape(E_PAD, 1)).reshape(128 * 128)[:N]
    rc = (1.0 / jnp.maximum(cnt, 1.0)).reshape(N, 1)

    y2, r2 = pl.pallas_call(
        _tc1,
        out_shape=[jax.ShapeDtypeStruct((N, D), f32),
                   jax.ShapeDtypeStruct((N, D), f32)],
    )(s1p, rc, x, W1_l, b1_l.reshape(1, -1), W1_r, W2_l, W2_r)

    s2p = _first(_agg(y2, src2d, dst2d))

    out = pl.pallas_call(
        _tc2,
        out_shape=jax.ShapeDtypeStruct((N, D), f32),
    )(s2p, rc, r2, b2_l.reshape(1, -1), Wl1, bl1.reshape(1, -1),
      Wl2, bl2.reshape(1, -1))
    return out


# speedup vs baseline: 4.5112x; 1.1053x over previous
import jax
import jax.numpy as jnp
from jax import lax
from jax.experimental import pallas as pl
from jax.experimental.pallas import tpu as pltpu
from jax.experimental.pallas import tpu_sc as plsc

N = 10000
E = 320000
D = 128

NC = 2
NS = 16
NW = NC * NS

EDGES_PER_TILE = E // NW
TILE_PAD = 10240
CHUNK = 64
CHUNKS = TILE_PAD // CHUNK
QCHUNKS = 32
PHASES = CHUNKS // QCHUNKS
NBUF = 4
N_ACC = 10240
TRASH = N
STRIPE = N_ACC // NS
ZROWS = 16
LAST_OUT = N - (NS - 1) * STRIPE

f32 = jnp.float32
i32 = jnp.int32


def _fill2d(ref, rows, cols, value):
    vals = jnp.full((16,), value, f32)

    def body(i, _):
        def body2(j, __):
            ref[i, pl.ds(j * 16, 16)] = vals
            return 0
        return lax.fori_loop(0, cols // 16, body2, 0)

    lax.fori_loop(0, rows, body, 0)


def _make_agg():
    mesh = plsc.VectorSubcoreMesh(core_axis_name="c", subcore_axis_name="s")
    out_type = [jax.ShapeDtypeStruct((NC, N, D), f32)]
    scratch = [
        pltpu.VMEM((QCHUNKS, CHUNK), i32),
        pltpu.VMEM((QCHUNKS, CHUNK), i32),
    ] + [pltpu.VMEM((CHUNK, D), f32) for _ in range(NBUF)] + [
        pltpu.VMEM((ZROWS, D), f32),
        pltpu.VMEM_SHARED((N_ACC, D), f32),
    ] + [pltpu.SemaphoreType.DMA for _ in range(2 * NBUF)]

    def body(table, src2d, dst2d, sum_out, src_v, dst_v, *rest):
        rows = rest[:NBUF]
        zbuf = rest[NBUF]
        accum = rest[NBUF + 1]
        sem_g = rest[NBUF + 2:NBUF + 2 + NBUF]
        sem_s = rest[NBUF + 2 + NBUF:]
        c = lax.axis_index("c")
        s = lax.axis_index("s")
        w = c * NS + s

        _fill2d(zbuf, ZROWS, D, 0.0)
        row0 = s * STRIPE
        for k in range(STRIPE // ZROWS):
            pltpu.sync_copy(zbuf, accum.at[pl.ds(row0 + k * ZROWS, ZROWS)])
        plsc.subcore_barrier()

        def phase(p, _):
            base = w * CHUNKS + p * QCHUNKS
            pltpu.sync_copy(src2d.at[pl.ds(base, QCHUNKS)], src_v)
            pltpu.sync_copy(dst2d.at[pl.ds(base, QCHUNKS)], dst_v)

            def group(g, __):
                c0 = g * NBUF
                gd = [pltpu.async_copy(table.at[src_v.at[c0 + b]], rows[b],
                                       sem_g[b]) for b in range(NBUF)]
                sd = []
                for b in range(NBUF):
                    gd[b].wait()
                    sd.append(pltpu.async_copy(rows[b],
                                               accum.at[dst_v.at[c0 + b]],
                                               sem_s[b], add=True))
                for b in range(NBUF):
                    sd[b].wait()
                return 0

            return lax.fori_loop(0, QCHUNKS // NBUF, group, 0)

        lax.fori_loop(0, PHASES, phase, 0)
        plsc.subcore_barrier()

        @pl.when(s < NS - 1)
        def _():
            pltpu.sync_copy(accum.at[pl.ds(row0, STRIPE)],
                            sum_out.at[c, pl.ds(row0, STRIPE)])

        @pl.when(s == NS - 1)
        def _():
            pltpu.sync_copy(accum.at[pl.ds(row0, LAST_OUT)],
                            sum_out.at[c, pl.ds(row0, LAST_OUT)])

    return pl.kernel(body, out_type=out_type, mesh=mesh,
                     scratch_types=scratch)


E_PAD = NW * TILE_PAD
CBS = 1024


def _cnt_body(dst_ref, cnt_ref):
    step = pl.program_id(0)

    @pl.when(step == 0)
    def _():
        cnt_ref[...] = jnp.zeros((128, 128), f32)

    d = dst_ref[...]
    lo = d & 127
    hi = d >> 7
    lane = lax.iota(i32, 128).reshape(1, 128)
    mlo = (lo == lane).astype(f32)
    mhi = (hi == lane).astype(f32)
    cnt_ref[...] += lax.dot_general(
        mhi, mlo, (((0,), (0,)), ((), ())), preferred_element_type=f32)


_count_tc = pl.pallas_call(
    _cnt_body,
    grid=(E_PAD // CBS,),
    in_specs=[pl.BlockSpec((CBS, 1), lambda i: (i, 0))],
    out_specs=pl.BlockSpec((128, 128), lambda i: (0, 0)),
    out_shape=jax.ShapeDtypeStruct((128, 128), f32),
)


_agg = _make_agg()


def _first(res):
    return res[0] if isinstance(res, (list, tuple)) else res


def _dotT(a, b):
    return lax.dot_general(a, b, (((1,), (1,)), ((), ())),
                           preferred_element_type=f32)


def _tc1(s1p_ref, rc_ref, x_ref, w1l_ref, b1l_ref, w1r_ref, w2l_ref,
         w2r_ref, y2_ref, r2_ref):
    rc = rc_ref[...]
    mean1 = (s1p_ref[0] + s1p_ref[1]) * rc
    h1 = _dotT(mean1, w1l_ref[...]) + b1l_ref[...] + _dotT(x_ref[...],
                                                           w1r_ref[...])
    h1 = jnp.maximum(h1, 0.0)
    y2_ref[...] = _dotT(h1, w2l_ref[...])
    r2_ref[...] = _dotT(h1, w2r_ref[...])


def _tc2(s2p_ref, rc_ref, r2_ref, b2l_ref, wl1_ref, bl1_ref, wl2_ref,
         bl2_ref, out_ref):
    rc = rc_ref[...]
    mean2 = (s2p_ref[0] + s2p_ref[1]) * rc
    h2 = jnp.maximum(mean2 + b2l_ref[...] + r2_ref[...], 0.0)
    h3 = _dotT(h2, wl1_ref[...]) + bl1_ref[...]
    out_ref[...] = _dotT(h3, wl2_ref[...]) + bl2_ref[...]


@jax.jit
def kernel(x, edge_index, W1_l, b1_l, W1_r, W2_l, b2_l, W2_r, Wl1, bl1,
           Wl2, bl2):
    pad = TILE_PAD - EDGES_PER_TILE
    src2d = jnp.pad(edge_index[0].reshape(NW, EDGES_PER_TILE),
                    ((0, 0), (0, pad)),
                    constant_values=0).reshape(NW * CHUNKS, CHUNK)
    dst2d = jnp.pad(edge_index[1].reshape(NW, EDGES_PER_TILE),
                    ((0, 0), (0, pad)),
                    constant_values=TRASH).reshape(NW * CHUNKS, CHUNK)

    s1p = _first(_agg(x, src2d, dst2d))
    cnt = _count_tc(dst2d.reshape(E_PAD, 1)).reshape(128 * 128)[:N]
    rc = (1.0 / jnp.maximum(cnt, 1.0)).reshape(N, 1)

    y2, r2 = pl.pallas_call(
        _tc1,
        out_shape=[jax.ShapeDtypeStruct((N, D), f32),
                   jax.ShapeDtypeStruct((N, D), f32)],
    )(s1p, rc, x, W1_l, b1_l.reshape(1, -1), W1_r, W2_l, W2_r)

    s2p = _first(_agg(y2, src2d, dst2d))

    out = pl.pallas_call(
        _tc2,
        out_shape=jax.ShapeDtypeStruct((N, D), f32),
    )(s2p, rc, r2, b2_l.reshape(1, -1), Wl1, bl1.reshape(1, -1),
      Wl2, bl2.reshape(1, -1))
    return out


# speedup vs baseline: 4.5191x; 1.0018x over previous
import jax
import jax.numpy as jnp
from jax import lax
from jax.experimental import pallas as pl
from jax.experimental.pallas import tpu as pltpu
from jax.experimental.pallas import tpu_sc as plsc

N = 10000
E = 320000
D = 128

NC = 2
NS = 16
NW = NC * NS

EDGES_PER_TILE = E // NW
TILE_PAD = 10240
CHUNK = 128
CHUNKS = TILE_PAD // CHUNK
QCHUNKS = 16
PHASES = CHUNKS // QCHUNKS
NBUF = 2
N_ACC = 10240
TRASH = N
STRIPE = N_ACC // NS
ZROWS = 16
LAST_OUT = N - (NS - 1) * STRIPE

f32 = jnp.float32
i32 = jnp.int32


def _fill2d(ref, rows, cols, value):
    vals = jnp.full((16,), value, f32)

    def body(i, _):
        def body2(j, __):
            ref[i, pl.ds(j * 16, 16)] = vals
            return 0
        return lax.fori_loop(0, cols // 16, body2, 0)

    lax.fori_loop(0, rows, body, 0)


def _make_agg():
    mesh = plsc.VectorSubcoreMesh(core_axis_name="c", subcore_axis_name="s")
    out_type = [jax.ShapeDtypeStruct((NC, N, D), f32)]
    scratch = [
        pltpu.VMEM((QCHUNKS, CHUNK), i32),
        pltpu.VMEM((QCHUNKS, CHUNK), i32),
    ] + [pltpu.VMEM((CHUNK, D), f32) for _ in range(NBUF)] + [
        pltpu.VMEM((ZROWS, D), f32),
        pltpu.VMEM_SHARED((N_ACC, D), f32),
    ] + [pltpu.SemaphoreType.DMA for _ in range(2 * NBUF)]

    def body(table, src2d, dst2d, sum_out, src_v, dst_v, *rest):
        rows = rest[:NBUF]
        zbuf = rest[NBUF]
        accum = rest[NBUF + 1]
        sem_g = rest[NBUF + 2:NBUF + 2 + NBUF]
        sem_s = rest[NBUF + 2 + NBUF:]
        c = lax.axis_index("c")
        s = lax.axis_index("s")
        w = c * NS + s

        _fill2d(zbuf, ZROWS, D, 0.0)
        row0 = s * STRIPE
        for k in range(STRIPE // ZROWS):
            pltpu.sync_copy(zbuf, accum.at[pl.ds(row0 + k * ZROWS, ZROWS)])
        plsc.subcore_barrier()

        def phase(p, _):
            base = w * CHUNKS + p * QCHUNKS
            pltpu.sync_copy(src2d.at[pl.ds(base, QCHUNKS)], src_v)
            pltpu.sync_copy(dst2d.at[pl.ds(base, QCHUNKS)], dst_v)

            def group(g, __):
                c0 = g * NBUF
                gd = [pltpu.async_copy(table.at[src_v.at[c0 + b]], rows[b],
                                       sem_g[b]) for b in range(NBUF)]
                sd = []
                for b in range(NBUF):
                    gd[b].wait()
                    sd.append(pltpu.async_copy(rows[b],
                                               accum.at[dst_v.at[c0 + b]],
                                               sem_s[b], add=True))
                for b in range(NBUF):
                    sd[b].wait()
                return 0

            return lax.fori_loop(0, QCHUNKS // NBUF, group, 0)

        lax.fori_loop(0, PHASES, phase, 0)
        plsc.subcore_barrier()

        @pl.when(s < NS - 1)
        def _():
            pltpu.sync_copy(accum.at[pl.ds(row0, STRIPE)],
                            sum_out.at[c, pl.ds(row0, STRIPE)])

        @pl.when(s == NS - 1)
        def _():
            pltpu.sync_copy(accum.at[pl.ds(row0, LAST_OUT)],
                            sum_out.at[c, pl.ds(row0, LAST_OUT)])

    return pl.kernel(body, out_type=out_type, mesh=mesh,
                     scratch_types=scratch)


E_PAD = NW * TILE_PAD
CBS = 1024


def _cnt_body(dst_ref, cnt_ref):
    step = pl.program_id(0)

    @pl.when(step == 0)
    def _():
        cnt_ref[...] = jnp.zeros((128, 128), f32)

    d = dst_ref[...]
    lo = d & 127
    hi = d >> 7
    lane = lax.iota(i32, 128).reshape(1, 128)
    mlo = (lo == lane).astype(f32)
    mhi = (hi == lane).astype(f32)
    cnt_ref[...] += lax.dot_general(
        mhi, mlo, (((0,), (0,)), ((), ())), preferred_element_type=f32)


_count_tc = pl.pallas_call(
    _cnt_body,
    grid=(E_PAD // CBS,),
    in_specs=[pl.BlockSpec((CBS, 1), lambda i: (i, 0))],
    out_specs=pl.BlockSpec((128, 128), lambda i: (0, 0)),
    out_shape=jax.ShapeDtypeStruct((128, 128), f32),
)


_agg = _make_agg()


def _first(res):
    return res[0] if isinstance(res, (list, tuple)) else res


def _dotT(a, b):
    return lax.dot_general(a, b, (((1,), (1,)), ((), ())),
                           preferred_element_type=f32)


def _tc1(s1p_ref, rc_ref, x_ref, w1l_ref, b1l_ref, w1r_ref, w2l_ref,
         w2r_ref, y2_ref, r2_ref):
    rc = rc_ref[...]
    mean1 = (s1p_ref[0] + s1p_ref[1]) * rc
    h1 = _dotT(mean1, w1l_ref[...]) + b1l_ref[...] + _dotT(x_ref[...],
                                                           w1r_ref[...])
    h1 = jnp.maximum(h1, 0.0)
    y2_ref[...] = _dotT(h1, w2l_ref[...])
    r2_ref[...] = _dotT(h1, w2r_ref[...])


def _tc2(s2p_ref, rc_ref, r2_ref, b2l_ref, wl1_ref, bl1_ref, wl2_ref,
         bl2_ref, out_ref):
    rc = rc_ref[...]
    mean2 = (s2p_ref[0] + s2p_ref[1]) * rc
    h2 = jnp.maximum(mean2 + b2l_ref[...] + r2_ref[...], 0.0)
    h3 = _dotT(h2, wl1_ref[...]) + bl1_ref[...]
    out_ref[...] = _dotT(h3, wl2_ref[...]) + bl2_ref[...]


@jax.jit
def kernel(x, edge_index, W1_l, b1_l, W1_r, W2_l, b2_l, W2_r, Wl1, bl1,
           Wl2, bl2):
    pad = TILE_PAD - EDGES_PER_TILE
    src2d = jnp.pad(edge_index[0].reshape(NW, EDGES_PER_TILE),
                    ((0, 0), (0, pad)),
                    constant_values=0).reshape(NW * CHUNKS, CHUNK)
    dst2d = jnp.pad(edge_index[1].reshape(NW, EDGES_PER_TILE),
                    ((0, 0), (0, pad)),
                    constant_values=TRASH).reshape(NW * CHUNKS, CHUNK)

    s1p = _first(_agg(x, src2d, dst2d))
    cnt = _count_tc(dst2d.reshape(E_PAD, 1)).reshape(128 * 128)[:N]
    rc = (1.0 / jnp.maximum(cnt, 1.0)).reshape(N, 1)

    y2, r2 = pl.pallas_call(
        _tc1,
        out_shape=[jax.ShapeDtypeStruct((N, D), f32),
                   jax.ShapeDtypeStruct((N, D), f32)],
    )(s1p, rc, x, W1_l, b1_l.reshape(1, -1), W1_r, W2_l, W2_r)

    s2p = _first(_agg(y2, src2d, dst2d))

    out = pl.pallas_call(
        _tc2,
        out_shape=jax.ShapeDtypeStruct((N, D), f32),
    )(s2p, rc, r2, b2_l.reshape(1, -1), Wl1, bl1.reshape(1, -1),
      Wl2, bl2.reshape(1, -1))
    return out


# speedup vs baseline: 4.6942x; 1.0387x over previous
import jax
import jax.numpy as jnp
from jax import lax
from jax.experimental import pallas as pl
from jax.experimental.pallas import tpu as pltpu
from jax.experimental.pallas import tpu_sc as plsc

N = 10000
E = 320000
D = 128

NC = 2
NS = 16
NW = NC * NS

EDGES_PER_TILE = E // NW
TILE_PAD = 10240
CHUNK = 64
CHUNKS = TILE_PAD // CHUNK
QCHUNKS = 32
PHASES = CHUNKS // QCHUNKS
NBUF = 4
N_ACC = 10240
TRASH = N
STRIPE = N_ACC // NS
ZROWS = 16
LAST_OUT = N - (NS - 1) * STRIPE

f32 = jnp.float32
i32 = jnp.int32


def _fill2d(ref, rows, cols, value):
    vals = jnp.full((16,), value, f32)

    def body(i, _):
        def body2(j, __):
            ref[i, pl.ds(j * 16, 16)] = vals
            return 0
        return lax.fori_loop(0, cols // 16, body2, 0)

    lax.fori_loop(0, rows, body, 0)


def _make_agg():
    mesh = plsc.VectorSubcoreMesh(core_axis_name="c", subcore_axis_name="s")
    out_type = [jax.ShapeDtypeStruct((NC, N, D), f32)]
    scratch = [
        pltpu.VMEM((QCHUNKS, CHUNK), i32),
        pltpu.VMEM((QCHUNKS, CHUNK), i32),
    ] + [pltpu.VMEM((CHUNK, D), f32) for _ in range(NBUF)] + [
        pltpu.VMEM((ZROWS, D), f32),
        pltpu.VMEM_SHARED((N_ACC, D), f32),
    ] + [pltpu.SemaphoreType.DMA for _ in range(2 * NBUF)]

    def body(table, src2d, dst2d, sum_out, src_v, dst_v, *rest):
        rows = rest[:NBUF]
        zbuf = rest[NBUF]
        accum = rest[NBUF + 1]
        sem_g = rest[NBUF + 2:NBUF + 2 + NBUF]
        sem_s = rest[NBUF + 2 + NBUF:]
        c = lax.axis_index("c")
        s = lax.axis_index("s")
        w = c * NS + s

        _fill2d(zbuf, ZROWS, D, 0.0)
        row0 = s * STRIPE
        for k in range(STRIPE // ZROWS):
            pltpu.sync_copy(zbuf, accum.at[pl.ds(row0 + k * ZROWS, ZROWS)])
        plsc.subcore_barrier()

        def phase(p, _):
            base = w * CHUNKS + p * QCHUNKS
            pltpu.sync_copy(src2d.at[pl.ds(base, QCHUNKS)], src_v)
            pltpu.sync_copy(dst2d.at[pl.ds(base, QCHUNKS)], dst_v)

            gd = {}
            sd = {}
            for j in range(QCHUNKS):
                if j >= NBUF:
                    sd[j - NBUF].wait()
                gd[j] = pltpu.async_copy(table.at[src_v.at[j]],
                                         rows[j % NBUF], sem_g[j % NBUF])
                jj = j - (NBUF - 1)
                if jj >= 0:
                    gd[jj].wait()
                    sd[jj] = pltpu.async_copy(rows[jj % NBUF],
                                              accum.at[dst_v.at[jj]],
                                              sem_s[jj % NBUF], add=True)
            for jj in range(QCHUNKS - NBUF + 1, QCHUNKS):
                gd[jj].wait()
                sd[jj] = pltpu.async_copy(rows[jj % NBUF],
                                          accum.at[dst_v.at[jj]],
                                          sem_s[jj % NBUF], add=True)
            for j in range(QCHUNKS - NBUF, QCHUNKS):
                sd[j].wait()
            return 0

        lax.fori_loop(0, PHASES, phase, 0)
        plsc.subcore_barrier()

        @pl.when(s < NS - 1)
        def _():
            pltpu.sync_copy(accum.at[pl.ds(row0, STRIPE)],
                            sum_out.at[c, pl.ds(row0, STRIPE)])

        @pl.when(s == NS - 1)
        def _():
            pltpu.sync_copy(accum.at[pl.ds(row0, LAST_OUT)],
                            sum_out.at[c, pl.ds(row0, LAST_OUT)])

    return pl.kernel(body, out_type=out_type, mesh=mesh,
                     scratch_types=scratch)


E_PAD = NW * TILE_PAD
CBS = 1024


def _cnt_body(dst_ref, cnt_ref):
    step = pl.program_id(0)

    @pl.when(step == 0)
    def _():
        cnt_ref[...] = jnp.zeros((128, 128), f32)

    d = dst_ref[...]
    lo = d & 127
    hi = d >> 7
    lane = lax.iota(i32, 128).reshape(1, 128)
    mlo = (lo == lane).astype(f32)
    mhi = (hi == lane).astype(f32)
    cnt_ref[...] += lax.dot_general(
        mhi, mlo, (((0,), (0,)), ((), ())), preferred_element_type=f32)


_count_tc = pl.pallas_call(
    _cnt_body,
    grid=(E_PAD // CBS,),
    in_specs=[pl.BlockSpec((CBS, 1), lambda i: (i, 0))],
    out_specs=pl.BlockSpec((128, 128), lambda i: (0, 0)),
    out_shape=jax.ShapeDtypeStruct((128, 128), f32),
)


_agg = _make_agg()


def _first(res):
    return res[0] if isinstance(res, (list, tuple)) else res


def _dotT(a, b):
    return lax.dot_general(a, b, (((1,), (1,)), ((), ())),
                           preferred_element_type=f32)


def _tc1(s1p_ref, rc_ref, x_ref, w1l_ref, b1l_ref, w1r_ref, w2l_ref,
         w2r_ref, y2_ref, r2_ref):
    rc = rc_ref[...]
    mean1 = (s1p_ref[0] + s1p_ref[1]) * rc
    h1 = _dotT(mean1, w1l_ref[...]) + b1l_ref[...] + _dotT(x_ref[...],
                                                           w1r_ref[...])
    h1 = jnp.maximum(h1, 0.0)
    y2_ref[...] = _dotT(h1, w2l_ref[...])
    r2_ref[...] = _dotT(h1, w2r_ref[...])


def _tc2(s2p_ref, rc_ref, r2_ref, b2l_ref, wl1_ref, bl1_ref, wl2_ref,
         bl2_ref, out_ref):
    rc = rc_ref[...]
    mean2 = (s2p_ref[0] + s2p_ref[1]) * rc
    h2 = jnp.maximum(mean2 + b2l_ref[...] + r2_ref[...], 0.0)
    h3 = _dotT(h2, wl1_ref[...]) + bl1_ref[...]
    out_ref[...] = _dotT(h3, wl2_ref[...]) + bl2_ref[...]


@jax.jit
def kernel(x, edge_index, W1_l, b1_l, W1_r, W2_l, b2_l, W2_r, Wl1, bl1,
           Wl2, bl2):
    pad = TILE_PAD - EDGES_PER_TILE
    src2d = jnp.pad(edge_index[0].reshape(NW, EDGES_PER_TILE),
                    ((0, 0), (0, pad)),
                    constant_values=0).reshape(NW * CHUNKS, CHUNK)
    dst2d = jnp.pad(edge_index[1].reshape(NW, EDGES_PER_TILE),
                    ((0, 0), (0, pad)),
                    constant_values=TRASH).reshape(NW * CHUNKS, CHUNK)

    s1p = _first(_agg(x, src2d, dst2d))
    cnt = _count_tc(dst2d.reshape(E_PAD, 1)).reshape(128 * 128)[:N]
    rc = (1.0 / jnp.maximum(cnt, 1.0)).reshape(N, 1)

    y2, r2 = pl.pallas_call(
        _tc1,
        out_shape=[jax.ShapeDtypeStruct((N, D), f32),
                   jax.ShapeDtypeStruct((N, D), f32)],
    )(s1p, rc, x, W1_l, b1_l.reshape(1, -1), W1_r, W2_l, W2_r)

    s2p = _first(_agg(y2, src2d, dst2d))

    out = pl.pallas_call(
        _tc2,
        out_shape=jax.ShapeDtypeStruct((N, D), f32),
    )(s2p, rc, r2, b2_l.reshape(1, -1), Wl1, bl1.reshape(1, -1),
      Wl2, bl2.reshape(1, -1))
    return out
